# baseline (speedup 1.0000x reference)
import functools

import jax
import jax.numpy as jnp
from jax import lax
from jax.experimental import pallas as pl
from jax.experimental.pallas import tpu as pltpu
from jax.experimental.pallas import tpu_sc as plsc

B = 128
V = 100000
L = 16
ROW_BLK = 8


def _gather_body(logits1d_hbm, tok_hbm, x_hbm, idx_v, x_v, sem):
    cid = lax.axis_index("c")
    sid = lax.axis_index("s")
    wid = sid * 2 + cid

    @pl.when(wid < B // L)
    def _():
        base = wid * L
        pltpu.sync_copy(tok_hbm.at[pl.ds(base, L)], idx_v)
        t = idx_v[...]
        lane_id = lax.iota(jnp.int32, L)
        flat = (base + lane_id) * V + t
        pltpu.async_copy(logits1d_hbm.at[flat], x_v, sem).wait()
        pltpu.sync_copy(x_v, x_hbm.at[pl.ds(base, L)])


@functools.cache
def _make_gather_call():
    return pl.kernel(
        _gather_body,
        out_type=jax.ShapeDtypeStruct((B,), jnp.float32),
        mesh=plsc.VectorSubcoreMesh(core_axis_name="c", subcore_axis_name="s"),
        scratch_types=[
            pltpu.VMEM((L,), jnp.int32),
            pltpu.VMEM((L,), jnp.float32),
            pltpu.SemaphoreType.DMA,
        ],
    )


def _count_body(x_ref, logits_ref, out_ref):
    x = x_ref[...]
    blk = logits_ref[...]
    out_ref[...] = jnp.sum((blk > x).astype(jnp.int32), axis=1, keepdims=True)


_count_call = pl.pallas_call(
    _count_body,
    grid=(B // ROW_BLK,),
    in_specs=[
        pl.BlockSpec((ROW_BLK, 1), lambda i: (i, 0)),
        pl.BlockSpec((ROW_BLK, V), lambda i: (i, 0)),
    ],
    out_specs=pl.BlockSpec((ROW_BLK, 1), lambda i: (i, 0)),
    out_shape=jax.ShapeDtypeStruct((B, 1), jnp.int32),
)


def kernel(logits, token_ids):
    tok = token_ids.astype(jnp.int32)
    logits1d = logits.reshape(B * V)
    x = _make_gather_call()(logits1d, tok)
    counts = _count_call(x.reshape(B, 1), logits)
    return counts.reshape(B).astype(jnp.int64)


# speedup vs baseline: 2.1325x; 2.1325x over previous
import functools

import jax
import jax.numpy as jnp
from jax import lax
from jax.experimental import pallas as pl
from jax.experimental.pallas import tpu as pltpu
from jax.experimental.pallas import tpu_sc as plsc

B = 128
V = 100000
L = 16
ROW_BLK = 8


def _gather_body(logits1d_hbm, tok_hbm, x_hbm, idx_v, x_v, sem):
    cid = lax.axis_index("c")
    sid = lax.axis_index("s")
    wid = sid * 2 + cid

    @pl.when(wid < B // L)
    def _():
        base = wid * L
        pltpu.sync_copy(tok_hbm.at[pl.ds(base, L)], idx_v)
        t = idx_v[...]
        lane_id = lax.iota(jnp.int32, L)
        flat = (base + lane_id) * V + t
        pltpu.async_copy(logits1d_hbm.at[flat], x_v, sem).wait()
        pltpu.sync_copy(x_v, x_hbm.at[pl.ds(base, L)])


@functools.cache
def _make_gather_call():
    return pl.kernel(
        _gather_body,
        out_type=jax.ShapeDtypeStruct((B,), jnp.float32),
        mesh=plsc.VectorSubcoreMesh(core_axis_name="c", subcore_axis_name="s"),
        scratch_types=[
            pltpu.VMEM((L,), jnp.int32),
            pltpu.VMEM((L,), jnp.float32),
            pltpu.SemaphoreType.DMA,
        ],
    )


def _count_body(x_ref, logits_ref, out_ref):
    x = x_ref[...]
    blk = logits_ref[...]
    out_ref[...] = jnp.sum((blk > x).astype(jnp.int32), axis=1, keepdims=True)


_count_call = pl.pallas_call(
    _count_body,
    grid=(B // ROW_BLK,),
    in_specs=[
        pl.BlockSpec((ROW_BLK, 1), lambda i: (i, 0)),
        pl.BlockSpec((ROW_BLK, V), lambda i: (i, 0)),
    ],
    out_specs=pl.BlockSpec((ROW_BLK, 1), lambda i: (i, 0)),
    out_shape=jax.ShapeDtypeStruct((B, 1), jnp.int32),
)


def kernel(logits, token_ids):
    tok = token_ids.astype(jnp.int32)
    x = jnp.take_along_axis(logits, tok[:, None], axis=1).reshape(B)
    counts = _count_call(x.reshape(B, 1), logits)
    return counts.reshape(B).astype(jnp.int64)


# speedup vs baseline: 2.3069x; 1.0818x over previous
import functools

import jax
import jax.numpy as jnp
from jax import lax
from jax.experimental import pallas as pl
from jax.experimental.pallas import tpu as pltpu
from jax.experimental.pallas import tpu_sc as plsc

B = 128
V = 100000
NOPS = 4
ROW_BLK = 8
RPO = B // NOPS
GRID = RPO // ROW_BLK


def _count_body(tok_ref, *refs):
    ins = refs[:NOPS]
    outs = refs[NOPS:]
    i = pl.program_id(0)
    col_iota = lax.broadcasted_iota(jnp.int32, (1, 128), 1)
    for k in range(NOPS):
        blk = ins[k][...]
        xs = []
        for r in range(ROW_BLK):
            t = tok_ref[k * RPO + i * ROW_BLK + r]
            start = pl.multiple_of(t & ~127, 128)
            sl = ins[k][pl.ds(r, 1), pl.ds(start, 128)]
            x = jnp.sum(
                jnp.where(col_iota == (t & 127), sl, 0.0), axis=1, keepdims=True
            )
            xs.append(x)
        xv = jnp.concatenate(xs, axis=0)
        outs[k][...] = jnp.sum((blk > xv).astype(jnp.int32), axis=1, keepdims=True)


@functools.cache
def _make_count_call():
    return pl.pallas_call(
        _count_body,
        grid=(GRID,),
        in_specs=[pl.BlockSpec(memory_space=pltpu.SMEM)]
        + [
            pl.BlockSpec((ROW_BLK, V), functools.partial(lambda k, i: (k * GRID + i, 0), k))
            for k in range(NOPS)
        ],
        out_specs=[
            pl.BlockSpec((ROW_BLK, 1), lambda i: (i, 0)) for _ in range(NOPS)
        ],
        out_shape=[
            jax.ShapeDtypeStruct((RPO, 1), jnp.int32) for _ in range(NOPS)
        ],
    )


def kernel(logits, token_ids):
    tok = token_ids.astype(jnp.int32)
    parts = _make_count_call()(tok, *([logits] * NOPS))
    counts = jnp.concatenate(parts, axis=0).reshape(B)
    return counts.astype(jnp.int64)


# speedup vs baseline: 2.3379x; 1.0135x over previous
import functools

import jax
import jax.numpy as jnp
from jax import lax
from jax.experimental import pallas as pl
from jax.experimental.pallas import tpu as pltpu
from jax.experimental.pallas import tpu_sc as plsc

B = 128
V = 100000
BLK = 8
NBLK = B // BLK


def _count_body(tok_ref, hbm_ref, out_ref, *scratch):
    bufs = scratch[:NBLK]
    sems = scratch[NBLK:]
    descs = [
        pltpu.make_async_copy(
            hbm_ref.at[pl.ds(s * BLK, BLK), :], bufs[s], sems[s]
        )
        for s in range(NBLK)
    ]
    for d in descs:
        d.start()
    col_iota = lax.broadcasted_iota(jnp.int32, (1, 128), 1)
    for s in range(NBLK):
        descs[s].wait()
        blk = bufs[s][...]
        xs = []
        for r in range(BLK):
            t = tok_ref[s * BLK + r]
            start = pl.multiple_of(t & ~127, 128)
            sl = bufs[s][pl.ds(r, 1), pl.ds(start, 128)]
            x = jnp.sum(
                jnp.where(col_iota == (t & 127), sl, 0.0), axis=1, keepdims=True
            )
            xs.append(x)
        xv = jnp.concatenate(xs, axis=0)
        out_ref[pl.ds(s * BLK, BLK), :] = jnp.sum(
            (blk > xv).astype(jnp.int32), axis=1, keepdims=True
        )


@functools.cache
def _make_count_call():
    return pl.pallas_call(
        _count_body,
        in_specs=[
            pl.BlockSpec(memory_space=pltpu.SMEM),
            pl.BlockSpec(memory_space=pltpu.HBM),
        ],
        out_specs=pl.BlockSpec(memory_space=pltpu.VMEM),
        out_shape=jax.ShapeDtypeStruct((B, 1), jnp.int32),
        scratch_shapes=[pltpu.VMEM((BLK, V), jnp.float32) for _ in range(NBLK)]
        + [pltpu.SemaphoreType.DMA for _ in range(NBLK)],
    )


def kernel(logits, token_ids):
    tok = token_ids.astype(jnp.int32)
    counts = _make_count_call()(tok, logits)
    return counts.reshape(B).astype(jnp.int64)


# speedup vs baseline: 8.2352x; 3.5224x over previous
import functools

import jax
import jax.numpy as jnp
from jax import lax
from jax.experimental import pallas as pl
from jax.experimental.pallas import tpu as pltpu
from jax.experimental.pallas import tpu_sc as plsc

B = 128
V = 100000
NBLK = 20
CV = V // NBLK


def _count_body(tok_ref, hbm_ref, out_ref, win_ref, wsem, *scratch):
    bufs = scratch[:NBLK]
    sems = scratch[NBLK:]
    wdescs = []
    for b in range(B):
        t = tok_ref[b]
        d = pltpu.make_async_copy(
            hbm_ref.at[pl.ds(t, 1), :], win_ref.at[pl.ds(b, 1), :], wsem
        )
        d.start()
        wdescs.append(d)
    descs = [
        pltpu.make_async_copy(hbm_ref.at[pl.ds(j * CV, CV), :], bufs[j], sems[j])
        for j in range(NBLK)
    ]
    for d in descs:
        d.start()
    for d in wdescs:
        d.wait()
    lane = lax.broadcasted_iota(jnp.int32, (1, B), 1)
    x = jnp.zeros((1, B), jnp.float32)
    for b in range(B):
        x = x + jnp.where(lane == b, win_ref[pl.ds(b, 1), :], 0.0)
    acc = jnp.zeros((1, B), jnp.int32)
    for j in range(NBLK):
        descs[j].wait()
        blk = bufs[j][...]
        acc = acc + jnp.sum((blk > x).astype(jnp.int32), axis=0, keepdims=True)
    out_ref[...] = acc


@functools.cache
def _make_count_call():
    return pl.pallas_call(
        _count_body,
        in_specs=[
            pl.BlockSpec(memory_space=pltpu.SMEM),
            pl.BlockSpec(memory_space=pltpu.HBM),
        ],
        out_specs=pl.BlockSpec(memory_space=pltpu.VMEM),
        out_shape=jax.ShapeDtypeStruct((1, B), jnp.int32),
        scratch_shapes=[pltpu.VMEM((B, B), jnp.float32), pltpu.SemaphoreType.DMA]
        + [pltpu.VMEM((CV, B), jnp.float32) for _ in range(NBLK)]
        + [pltpu.SemaphoreType.DMA for _ in range(NBLK)],
    )


def kernel(logits, token_ids):
    tok = token_ids.astype(jnp.int32)
    counts = _make_count_call()(tok, logits.T)
    return counts.reshape(B).astype(jnp.int64)
